# baseline (speedup 1.0000x reference)
import functools

import numpy as np
import jax
import jax.numpy as jnp
from jax import lax
from jax.experimental import pallas as pl
from jax.experimental.pallas import tpu as pltpu
from jax.experimental.pallas import tpu_sc as plsc

VOCAB = 100000
D_MODEL = 1024
MAX_LEN = 2048
BATCH = 4
SEQ = 2048

NC, NS = 2, 16
NW = NC * NS
TOTAL = BATCH * SEQ
PER_W = TOTAL // NW
CHUNK = 16
N_CHUNKS = PER_W // CHUNK
NR = 2
NO = 4
SUPER = 4
SCALE = float(D_MODEL) ** 0.5


def _make_pe(max_len, d_model):
    pe = np.zeros((max_len, d_model), dtype=np.float32)
    position = np.arange(0, max_len, dtype=np.float32)[:, None]
    div_term = np.exp(
        np.arange(0, d_model, 2, dtype=np.float32) * -(np.log(10000.0) / d_model))
    pe[:, 0::2] = np.sin(position * div_term)
    pe[:, 1::2] = np.cos(position * div_term)
    return pe


_PE = _make_pe(MAX_LEN, D_MODEL)


def _sc_embed(x_flat, table, pe):
    mesh = plsc.VectorSubcoreMesh(core_axis_name="c", subcore_axis_name="s")

    @functools.partial(
        pl.kernel,
        out_type=jax.ShapeDtypeStruct((TOTAL, D_MODEL), jnp.float32),
        mesh=mesh,
        scratch_types=[
            pltpu.VMEM((PER_W,), jnp.int32),
            [pltpu.VMEM((CHUNK, D_MODEL), jnp.float32) for _ in range(NR)],
            [pltpu.VMEM((CHUNK, D_MODEL), jnp.float32) for _ in range(NO)],
            [pltpu.SemaphoreType.DMA for _ in range(NR)],
            [pltpu.SemaphoreType.DMA for _ in range(NO)],
            [pltpu.SemaphoreType.DMA for _ in range(NO)],
        ],
    )
    def k(x_hbm, table_hbm, pe_hbm, out_hbm,
          idx_v, rows, obufs, gsems, psems, osems):
        wid = lax.axis_index("s") * NC + lax.axis_index("c")
        base = wid * PER_W
        s0 = base % SEQ

        pltpu.sync_copy(x_hbm.at[pl.ds(base, PER_W)], idx_v)

        def fire_gather(c, br):
            pltpu.async_copy(
                table_hbm.at[idx_v.at[pl.ds(c * CHUNK, CHUNK)]],
                rows[br], gsems[br])

        def wait_gather(c, br):
            pltpu.make_async_copy(
                table_hbm.at[idx_v.at[pl.ds(c * CHUNK, CHUNK)]],
                rows[br], gsems[br]).wait()

        def fire_pe(c, bo):
            pltpu.async_copy(
                pe_hbm.at[pl.ds(s0 + c * CHUNK, CHUNK)], obufs[bo], psems[bo])

        def wait_pe(c, bo):
            pltpu.make_async_copy(
                pe_hbm.at[pl.ds(s0 + c * CHUNK, CHUNK)], obufs[bo],
                psems[bo]).wait()

        def fire_out(c, bo):
            pltpu.async_copy(
                obufs[bo], out_hbm.at[pl.ds(base + c * CHUNK, CHUNK)],
                osems[bo])

        def wait_out(c, bo):
            pltpu.make_async_copy(
                obufs[bo], out_hbm.at[pl.ds(base + c * CHUNK, CHUNK)],
                osems[bo]).wait()

        fire_gather(0, 0)
        fire_gather(1, 1)
        fire_pe(0, 0)
        fire_pe(1, 1)

        def super_body(g, _):
            c0 = g * SUPER
            for j in range(SUPER):
                c = c0 + j
                br, bo = j % NR, j % NO

                @pl.when(c >= NO - 2)
                def _():
                    wait_out(c - (NO - 2), (j + 2) % NO)

                @pl.when(c + 2 < N_CHUNKS)
                def _():
                    fire_pe(c + 2, (j + 2) % NO)

                wait_gather(c, br)
                wait_pe(c, bo)

                @plsc.parallel_loop(0, CHUNK, unroll=2)
                def row_body(r):
                    for q in range(D_MODEL // 16):
                        sl = pl.ds(q * 16, 16)
                        plsc.addupdate(
                            obufs[bo].at[r, sl], rows[br][r, sl] * SCALE)

                fire_out(c, bo)

                @pl.when(c + 2 < N_CHUNKS)
                def _():
                    fire_gather(c + 2, br)
            return 0

        lax.fori_loop(0, N_CHUNKS // SUPER, super_body, 0)

        for c in range(N_CHUNKS - 2, N_CHUNKS):
            wait_out(c, c % NO)

    return k(x_flat, table, pe)


def kernel(x, table):
    x_flat = jnp.reshape(x, (TOTAL,)).astype(jnp.int32)
    out = _sc_embed(x_flat, table, _PE)
    return jnp.reshape(out, (BATCH, SEQ, D_MODEL))


# speedup vs baseline: 1.2616x; 1.2616x over previous
import functools

import numpy as np
import jax
import jax.numpy as jnp
from jax import lax
from jax.experimental import pallas as pl
from jax.experimental.pallas import tpu as pltpu
from jax.experimental.pallas import tpu_sc as plsc

VOCAB = 100000
D_MODEL = 1024
MAX_LEN = 2048
BATCH = 4
SEQ = 2048

NC, NS = 2, 16
NW = NC * NS
TOTAL = BATCH * SEQ
PER_W = TOTAL // NW
CHUNK = 16
N_CHUNKS = PER_W // CHUNK
NR = 2
NO = 4
SUPER = 4
SCALE = float(D_MODEL) ** 0.5


def _make_pe(max_len, d_model):
    pe = np.zeros((max_len, d_model), dtype=np.float32)
    position = np.arange(0, max_len, dtype=np.float32)[:, None]
    div_term = np.exp(
        np.arange(0, d_model, 2, dtype=np.float32) * -(np.log(10000.0) / d_model))
    pe[:, 0::2] = np.sin(position * div_term)
    pe[:, 1::2] = np.cos(position * div_term)
    return pe


_PE = _make_pe(MAX_LEN, D_MODEL)


def _sc_embed(x_flat, table, pe):
    mesh = plsc.VectorSubcoreMesh(core_axis_name="c", subcore_axis_name="s")

    @functools.partial(
        pl.kernel,
        out_type=jax.ShapeDtypeStruct((TOTAL, D_MODEL), jnp.float32),
        mesh=mesh,
        scratch_types=[
            pltpu.VMEM((PER_W,), jnp.int32),
            [pltpu.VMEM((CHUNK, D_MODEL), jnp.float32) for _ in range(NR)],
            [pltpu.VMEM((CHUNK, D_MODEL), jnp.float32) for _ in range(NO)],
            [pltpu.SemaphoreType.DMA for _ in range(NR)],
            [pltpu.SemaphoreType.DMA for _ in range(NO)],
            [pltpu.SemaphoreType.DMA for _ in range(NO)],
        ],
    )
    def k(x_hbm, table_hbm, pe_hbm, out_hbm,
          idx_v, rows, obufs, gsems, psems, osems):
        wid = lax.axis_index("s") * NC + lax.axis_index("c")
        base = wid * PER_W
        s0 = base % SEQ

        pltpu.sync_copy(x_hbm.at[pl.ds(base, PER_W)], idx_v)

        def fire_gather(c, br):
            pltpu.async_copy(
                table_hbm.at[idx_v.at[pl.ds(c * CHUNK, CHUNK)]],
                rows[br], gsems[br])

        def wait_gather(c, br):
            pltpu.make_async_copy(
                table_hbm.at[idx_v.at[pl.ds(c * CHUNK, CHUNK)]],
                rows[br], gsems[br]).wait()

        def fire_pe(c, bo):
            pltpu.async_copy(
                pe_hbm.at[pl.ds(s0 + c * CHUNK, CHUNK)], obufs[bo], psems[bo])

        def wait_pe(c, bo):
            pltpu.make_async_copy(
                pe_hbm.at[pl.ds(s0 + c * CHUNK, CHUNK)], obufs[bo],
                psems[bo]).wait()

        def fire_out(c, bo):
            pltpu.async_copy(
                obufs[bo], out_hbm.at[pl.ds(base + c * CHUNK, CHUNK)],
                osems[bo])

        def wait_out(c, bo):
            pltpu.make_async_copy(
                obufs[bo], out_hbm.at[pl.ds(base + c * CHUNK, CHUNK)],
                osems[bo]).wait()

        fire_gather(0, 0)
        fire_gather(1, 1)
        fire_pe(0, 0)
        fire_pe(1, 1)

        def super_body(g, _):
            c0 = g * SUPER
            for j in range(SUPER):
                c = c0 + j
                br, bo = j % NR, j % NO

                @pl.when(c >= NO - 2)
                def _():
                    wait_out(c - (NO - 2), (j + 2) % NO)

                @pl.when(c + 2 < N_CHUNKS)
                def _():
                    fire_pe(c + 2, (j + 2) % NO)

                wait_gather(c, br)
                wait_pe(c, bo)

                @plsc.parallel_loop(0, CHUNK)
                def row_body(r):
                    for q in range(D_MODEL // 16):
                        sl = pl.ds(q * 16, 16)
                        plsc.addupdate(
                            obufs[bo].at[r, sl], rows[br][r, sl] * SCALE)

                fire_out(c, bo)

                @pl.when(c + 2 < N_CHUNKS)
                def _():
                    fire_gather(c + 2, br)
            return 0

        lax.fori_loop(0, N_CHUNKS // SUPER, super_body, 0)

        for c in range(N_CHUNKS - 2, N_CHUNKS):
            wait_out(c, c % NO)

    return k(x_flat, table, pe)


def kernel(x, table):
    x_flat = jnp.reshape(x, (TOTAL,)).astype(jnp.int32)
    out = _sc_embed(x_flat, table, _PE)
    return jnp.reshape(out, (BATCH, SEQ, D_MODEL))


# speedup vs baseline: 1.2946x; 1.0261x over previous
import functools

import numpy as np
import jax
import jax.numpy as jnp
from jax import lax
from jax.experimental import pallas as pl
from jax.experimental.pallas import tpu as pltpu
from jax.experimental.pallas import tpu_sc as plsc

VOCAB = 100000
D_MODEL = 1024
MAX_LEN = 2048
BATCH = 4
SEQ = 2048

NC, NS = 2, 16
NW = NC * NS
TOTAL = BATCH * SEQ
PER_W = TOTAL // NW
CHUNK = 16
N_CHUNKS = PER_W // CHUNK
NR = 2
NO = 4
SUPER = 4
SCALE = float(D_MODEL) ** 0.5


def _make_pe(max_len, d_model):
    pe = np.zeros((max_len, d_model), dtype=np.float32)
    position = np.arange(0, max_len, dtype=np.float32)[:, None]
    div_term = np.exp(
        np.arange(0, d_model, 2, dtype=np.float32) * -(np.log(10000.0) / d_model))
    pe[:, 0::2] = np.sin(position * div_term)
    pe[:, 1::2] = np.cos(position * div_term)
    return pe


_PE = _make_pe(MAX_LEN, D_MODEL)


def _sc_embed(x_flat, table, pe):
    mesh = plsc.VectorSubcoreMesh(core_axis_name="c", subcore_axis_name="s")

    @functools.partial(
        pl.kernel,
        out_type=jax.ShapeDtypeStruct((BATCH, SEQ, D_MODEL), jnp.float32),
        mesh=mesh,
        scratch_types=[
            pltpu.VMEM((PER_W,), jnp.int32),
            [pltpu.VMEM((CHUNK, D_MODEL), jnp.float32) for _ in range(NR)],
            [pltpu.VMEM((CHUNK, D_MODEL), jnp.float32) for _ in range(NO)],
            [pltpu.SemaphoreType.DMA for _ in range(NR)],
            [pltpu.SemaphoreType.DMA for _ in range(NO)],
            [pltpu.SemaphoreType.DMA for _ in range(NO)],
        ],
    )
    def k(x_hbm, table_hbm, pe_hbm, out_hbm,
          idx_v, rows, obufs, gsems, psems, osems):
        wid = lax.axis_index("s") * NC + lax.axis_index("c")
        b_idx = wid // (SEQ // PER_W)
        s0 = (wid % (SEQ // PER_W)) * PER_W

        pltpu.sync_copy(x_hbm.at[b_idx, pl.ds(s0, PER_W)], idx_v)

        def fire_gather(c, br):
            pltpu.async_copy(
                table_hbm.at[idx_v.at[pl.ds(c * CHUNK, CHUNK)]],
                rows[br], gsems[br])

        def wait_gather(c, br):
            pltpu.make_async_copy(
                table_hbm.at[idx_v.at[pl.ds(c * CHUNK, CHUNK)]],
                rows[br], gsems[br]).wait()

        def fire_pe(c, bo):
            pltpu.async_copy(
                pe_hbm.at[pl.ds(s0 + c * CHUNK, CHUNK)], obufs[bo], psems[bo])

        def wait_pe(c, bo):
            pltpu.make_async_copy(
                pe_hbm.at[pl.ds(s0 + c * CHUNK, CHUNK)], obufs[bo],
                psems[bo]).wait()

        def fire_out(c, bo):
            pltpu.async_copy(
                obufs[bo],
                out_hbm.at[b_idx, pl.ds(s0 + c * CHUNK, CHUNK)], osems[bo])

        def wait_out(c, bo):
            pltpu.make_async_copy(
                obufs[bo],
                out_hbm.at[b_idx, pl.ds(s0 + c * CHUNK, CHUNK)],
                osems[bo]).wait()

        fire_gather(0, 0)
        fire_gather(1, 1)
        fire_pe(0, 0)
        fire_pe(1, 1)

        def super_body(g, _):
            c0 = g * SUPER
            for j in range(SUPER):
                c = c0 + j
                br, bo = j % NR, j % NO

                @pl.when(c >= NO - 2)
                def _():
                    wait_out(c - (NO - 2), (j + 2) % NO)

                @pl.when(c + 2 < N_CHUNKS)
                def _():
                    fire_pe(c + 2, (j + 2) % NO)

                wait_gather(c, br)
                wait_pe(c, bo)

                @plsc.parallel_loop(0, CHUNK)
                def row_body(r):
                    for q in range(D_MODEL // 16):
                        sl = pl.ds(q * 16, 16)
                        plsc.addupdate(
                            obufs[bo].at[r, sl], rows[br][r, sl] * SCALE)

                fire_out(c, bo)

                @pl.when(c + 2 < N_CHUNKS)
                def _():
                    fire_gather(c + 2, br)
            return 0

        lax.fori_loop(0, N_CHUNKS // SUPER, super_body, 0)

        for c in range(N_CHUNKS - 2, N_CHUNKS):
            wait_out(c, c % NO)

    return k(x_flat, table, pe)


def kernel(x, table):
    return _sc_embed(x.astype(jnp.int32), table, _PE)


# speedup vs baseline: 1.3818x; 1.0674x over previous
import functools

import numpy as np
import jax
import jax.numpy as jnp
from jax import lax
from jax.experimental import pallas as pl
from jax.experimental.pallas import tpu as pltpu
from jax.experimental.pallas import tpu_sc as plsc

VOCAB = 100000
D_MODEL = 1024
MAX_LEN = 2048
BATCH = 4
SEQ = 2048

NC, NS = 2, 16
NW = NC * NS
TOTAL = BATCH * SEQ
PER_W = TOTAL // NW
CHUNK = 16
N_CHUNKS = PER_W // CHUNK
NR = 2
NO = 4
NP = 2
SUPER = 4
SCALE = float(D_MODEL) ** 0.5
PE_WORDS = D_MODEL // 2


def _make_pe(max_len, d_model):
    pe = np.zeros((max_len, d_model), dtype=np.float32)
    position = np.arange(0, max_len, dtype=np.float32)[:, None]
    div_term = np.exp(
        np.arange(0, d_model, 2, dtype=np.float32) * -(np.log(10000.0) / d_model))
    pe[:, 0::2] = np.sin(position * div_term)
    pe[:, 1::2] = np.cos(position * div_term)
    return pe


def _pack_pe(pe):
    bits = pe.view(np.uint32)
    bf = ((bits + 0x7FFF + ((bits >> 16) & 1)) >> 16).astype(np.uint32)
    g = bf.reshape(pe.shape[0], D_MODEL // 32, 2, 16)
    words = g[:, :, 0, :] | (g[:, :, 1, :] << 16)
    return words.reshape(pe.shape[0], PE_WORDS).view(np.int32)


_PE_PACKED = _pack_pe(_make_pe(MAX_LEN, D_MODEL))


def _sc_embed(x, table, pe_pk):
    mesh = plsc.VectorSubcoreMesh(core_axis_name="c", subcore_axis_name="s")

    @functools.partial(
        pl.kernel,
        out_type=jax.ShapeDtypeStruct((BATCH, SEQ, D_MODEL), jnp.float32),
        mesh=mesh,
        scratch_types=[
            pltpu.VMEM((PER_W,), jnp.int32),
            [pltpu.VMEM((CHUNK, D_MODEL), jnp.float32) for _ in range(NR)],
            [pltpu.VMEM((CHUNK, D_MODEL), jnp.float32) for _ in range(NO)],
            [pltpu.VMEM((CHUNK, PE_WORDS), jnp.int32) for _ in range(NP)],
            [pltpu.SemaphoreType.DMA for _ in range(NR)],
            [pltpu.SemaphoreType.DMA for _ in range(NP)],
            [pltpu.SemaphoreType.DMA for _ in range(NO)],
        ],
    )
    def k(x_hbm, table_hbm, pe_hbm, out_hbm,
          idx_v, rows, obufs, pebs, gsems, psems, osems):
        wid = lax.axis_index("s") * NC + lax.axis_index("c")
        b_idx = wid // (SEQ // PER_W)
        s0 = (wid % (SEQ // PER_W)) * PER_W

        pltpu.sync_copy(x_hbm.at[b_idx, pl.ds(s0, PER_W)], idx_v)

        def fire_gather(c, br):
            pltpu.async_copy(
                table_hbm.at[idx_v.at[pl.ds(c * CHUNK, CHUNK)]],
                rows[br], gsems[br])

        def wait_gather(c, br):
            pltpu.make_async_copy(
                table_hbm.at[idx_v.at[pl.ds(c * CHUNK, CHUNK)]],
                rows[br], gsems[br]).wait()

        def fire_pe(c, bp):
            pltpu.async_copy(
                pe_hbm.at[pl.ds(s0 + c * CHUNK, CHUNK)], pebs[bp], psems[bp])

        def wait_pe(c, bp):
            pltpu.make_async_copy(
                pe_hbm.at[pl.ds(s0 + c * CHUNK, CHUNK)], pebs[bp],
                psems[bp]).wait()

        def fire_out(c, bo):
            pltpu.async_copy(
                obufs[bo],
                out_hbm.at[b_idx, pl.ds(s0 + c * CHUNK, CHUNK)], osems[bo])

        def wait_out(c, bo):
            pltpu.make_async_copy(
                obufs[bo],
                out_hbm.at[b_idx, pl.ds(s0 + c * CHUNK, CHUNK)],
                osems[bo]).wait()

        fire_gather(0, 0)
        fire_gather(1, 1)
        fire_pe(0, 0)
        fire_pe(1, 1)

        def super_body(g, _):
            c0 = g * SUPER
            for j in range(SUPER):
                c = c0 + j
                br, bo, bp = j % NR, j % NO, j % NP

                @pl.when(c >= NO)
                def _():
                    wait_out(c - NO, bo)

                wait_gather(c, br)
                wait_pe(c, bp)

                @plsc.parallel_loop(0, CHUNK)
                def row_body(r):
                    for q in range(D_MODEL // 32):
                        w = pebs[bp][r, pl.ds(q * 16, 16)]
                        lo = lax.bitcast_convert_type(w << 16, jnp.float32)
                        hi = lax.bitcast_convert_type(
                            w & jnp.int32(-65536), jnp.float32)
                        sl0 = pl.ds(q * 32, 16)
                        sl1 = pl.ds(q * 32 + 16, 16)
                        obufs[bo][r, sl0] = rows[br][r, sl0] * SCALE + lo
                        obufs[bo][r, sl1] = rows[br][r, sl1] * SCALE + hi

                fire_out(c, bo)

                @pl.when(c + 2 < N_CHUNKS)
                def _():
                    fire_pe(c + 2, bp)
                    fire_gather(c + 2, br)
            return 0

        lax.fori_loop(0, N_CHUNKS // SUPER, super_body, 0)

        for c in range(N_CHUNKS - NO, N_CHUNKS):
            wait_out(c, c % NO)

    return k(x, table, pe_pk)


def kernel(x, table):
    return _sc_embed(x.astype(jnp.int32), table, _PE_PACKED)


# speedup vs baseline: 1.4561x; 1.0537x over previous
import functools

import numpy as np
import jax
import jax.numpy as jnp
from jax import lax
from jax.experimental import pallas as pl
from jax.experimental.pallas import tpu as pltpu
from jax.experimental.pallas import tpu_sc as plsc

VOCAB = 100000
D_MODEL = 1024
MAX_LEN = 2048
BATCH = 4
SEQ = 2048

NC, NS = 2, 16
NW = NC * NS
SPAN = SEQ // NW
PER_W = BATCH * SPAN
CHUNK = 16
N_CHUNKS = PER_W // CHUNK
CPB = SPAN // CHUNK
NR = 4
SCALE = float(D_MODEL) ** 0.5
PE_WORDS = D_MODEL // 2


def _make_pe(max_len, d_model):
    pe = np.zeros((max_len, d_model), dtype=np.float32)
    position = np.arange(0, max_len, dtype=np.float32)[:, None]
    div_term = np.exp(
        np.arange(0, d_model, 2, dtype=np.float32) * -(np.log(10000.0) / d_model))
    pe[:, 0::2] = np.sin(position * div_term)
    pe[:, 1::2] = np.cos(position * div_term)
    return pe


def _pack_pe(pe):
    bits = pe.view(np.uint32)
    bf = ((bits + 0x7FFF + ((bits >> 16) & 1)) >> 16).astype(np.uint32)
    g = bf.reshape(pe.shape[0], D_MODEL // 32, 2, 16)
    words = g[:, :, 0, :] | (g[:, :, 1, :] << 16)
    return words.reshape(pe.shape[0], PE_WORDS).view(np.int32)


_PE_PACKED = _pack_pe(_make_pe(MAX_LEN, D_MODEL))


def _sc_embed(x, table, pe_pk):
    mesh = plsc.VectorSubcoreMesh(core_axis_name="c", subcore_axis_name="s")

    @functools.partial(
        pl.kernel,
        out_type=jax.ShapeDtypeStruct((BATCH, SEQ, D_MODEL), jnp.float32),
        mesh=mesh,
        scratch_types=[
            pltpu.VMEM((PER_W,), jnp.int32),
            pltpu.VMEM((SPAN, PE_WORDS), jnp.int32),
            [pltpu.VMEM((CHUNK, D_MODEL), jnp.float32) for _ in range(NR)],
            pltpu.SemaphoreType.DMA,
            pltpu.SemaphoreType.DMA,
            [pltpu.SemaphoreType.DMA for _ in range(NR)],
            [pltpu.SemaphoreType.DMA for _ in range(NR)],
        ],
    )
    def k(x_hbm, table_hbm, pe_hbm, out_hbm,
          idx_v, pe_res, rows, isem, psem, gsems, osems):
        wid = lax.axis_index("s") * NC + lax.axis_index("c")
        s0 = wid * SPAN

        pltpu.async_copy(pe_hbm.at[pl.ds(s0, SPAN)], pe_res, psem)
        for b in range(BATCH):
            pltpu.async_copy(
                x_hbm.at[b, pl.ds(s0, SPAN)],
                idx_v.at[pl.ds(b * SPAN, SPAN)], isem)
        for b in range(BATCH):
            pltpu.make_async_copy(
                x_hbm.at[b, pl.ds(s0, SPAN)],
                idx_v.at[pl.ds(b * SPAN, SPAN)], isem).wait()

        def fire_gather(c, br):
            pltpu.async_copy(
                table_hbm.at[idx_v.at[pl.ds(c * CHUNK, CHUNK)]],
                rows[br], gsems[br])

        def wait_gather(c, br):
            pltpu.make_async_copy(
                table_hbm.at[idx_v.at[pl.ds(c * CHUNK, CHUNK)]],
                rows[br], gsems[br]).wait()

        def fire_out(b, t, br):
            pltpu.async_copy(
                rows[br],
                out_hbm.at[b, pl.ds(s0 + t * CHUNK, CHUNK)], osems[br])

        def wait_out(b, t, br):
            pltpu.make_async_copy(
                rows[br],
                out_hbm.at[b, pl.ds(s0 + t * CHUNK, CHUNK)],
                osems[br]).wait()

        fire_gather(0, 0)
        fire_gather(1, 1)
        pltpu.make_async_copy(
            pe_hbm.at[pl.ds(s0, SPAN)], pe_res, psem).wait()

        def super_body(b, _):
            c0 = b * CPB
            for t in range(CPB):
                c = c0 + t
                br = t % NR
                wait_gather(c, br)

                @plsc.parallel_loop(0, CHUNK)
                def row_body(r):
                    for q in range(D_MODEL // 32):
                        w = pe_res[t * CHUNK + r, pl.ds(q * 16, 16)]
                        lo = lax.bitcast_convert_type(w << 16, jnp.float32)
                        hi = lax.bitcast_convert_type(
                            w & jnp.int32(-65536), jnp.float32)
                        sl0 = pl.ds(q * 32, 16)
                        sl1 = pl.ds(q * 32 + 16, 16)
                        rows[br][r, sl0] = rows[br][r, sl0] * SCALE + lo
                        rows[br][r, sl1] = rows[br][r, sl1] * SCALE + hi

                fire_out(b, t, br)

                @pl.when(c + 2 < N_CHUNKS)
                def _():
                    nb = (t + 2) % NR

                    @pl.when(c >= 2)
                    def _():
                        wait_out(b - 1 + (t + 2) // CPB,
                                 (t + 2) % CPB, nb)

                    fire_gather(c + 2, nb)
            return 0

        lax.fori_loop(0, BATCH, super_body, 0)

        for c in range(N_CHUNKS - NR, N_CHUNKS):
            wait_out(c // CPB, c % CPB, c % NR)

    return k(x, table, pe_pk)


def kernel(x, table):
    return _sc_embed(x.astype(jnp.int32), table, _PE_PACKED)
